# baseline (speedup 1.0000x reference)
import jax
import jax.numpy as jnp
from jax import lax
from jax.experimental import pallas as pl
from jax.experimental.pallas import tpu as pltpu
from jax.experimental.pallas import tpu_sc as plsc

VOCAB = 1000000
HIDDEN = 64
BATCH = 1024
SEQ = 200
EPS = 1e-05

NC, NS, L = 2, 16, 16
NW = NC * NS
N_ROWS = BATCH * SEQ
R_PER_W = N_ROWS // NW
CHUNK = 256
N_CHUNKS = R_PER_W // CHUNK
PW = 2 * HIDDEN
OPITCH = 129


def _rsqrt16(x):
    i = plsc.bitcast(x, jnp.int32)
    i = jnp.int32(0x5F3759DF) - lax.shift_right_logical(i, 1)
    y = plsc.bitcast(i, jnp.float32)
    h = x * 0.5
    y = y * (1.5 - h * y * y)
    y = y * (1.5 - h * y * y)
    return y


def _body(ids_ref, table_ref, tt_ref, gamma_ref, beta_ref, out_ref,
          idx_v, rows0, rows1, obuf, tv_v, gv_v, bv_v, gsem0, gsem1):
    wid = lax.axis_index("s") * NC + lax.axis_index("c")
    base = wid * R_PER_W

    pltpu.sync_copy(ids_ref.at[pl.ds(wid * R_PER_W, R_PER_W)], idx_v)
    pltpu.sync_copy(tt_ref.at[0], tv_v)
    pltpu.sync_copy(gamma_ref, gv_v)
    pltpu.sync_copy(beta_ref, bv_v)

    tvs = [tv_v[pl.ds(i * L, L)] for i in range(4)]
    gvs = [gv_v[pl.ds(i * L, L)] for i in range(4)]
    bvs = [bv_v[pl.ds(i * L, L)] for i in range(4)]

    rows = (rows0, rows1)
    gsems = (gsem0, gsem1)

    def start_gather(g, b):
        idx = idx_v.at[pl.ds(g * CHUNK, CHUNK)]
        pltpu.async_copy(table_ref.at[idx], rows[b], gsems[b])

    def process(g, b):
        buf = rows[b]
        idx = idx_v.at[pl.ds(g * CHUNK, CHUNK)]
        pltpu.make_async_copy(table_ref.at[idx], buf, gsems[b]).wait()

        lane = lax.iota(jnp.int32, L)
        thvs = [2 * k + lax.shift_right_logical(lane, 3) for k in range(4)]
        hsv = lane & 7

        @plsc.parallel_loop(0, CHUNK // L, unroll=1)
        def _grp(j):
            tblv = jnp.full((L,), j >> 3, dtype=jnp.int32)
            for r in range(L):
                i = j * L + r
                y = [buf[i, pl.ds(k * L, L)] + tvs[k] for k in range(4)]
                s = jnp.sum((y[0] + y[1]) + (y[2] + y[3]))
                q = jnp.sum((y[0] * y[0] + y[1] * y[1])
                            + (y[2] * y[2] + y[3] * y[3]))
                mean = s * (1.0 / HIDDEN)
                var = q * (1.0 / HIDDEN) - mean * mean
                mb = jnp.full((L,), mean, dtype=jnp.float32)
                rb = _rsqrt16(jnp.full((L,), var + EPS, dtype=jnp.float32))
                bsv = jnp.full((L,), (j & 7) * L + r, dtype=jnp.int32)
                for k in range(4):
                    plsc.store_scatter(obuf, [tblv, thvs[k], hsv, bsv],
                                       (y[k] - mb) * rb * gvs[k] + bvs[k])

        row0 = base + g * CHUNK
        s_idx = row0 // BATCH
        tb0 = (row0 % BATCH) // 128
        for tbl in range(2):
            pltpu.sync_copy(
                obuf.at[tbl, :, :, pl.ds(0, 128)],
                out_ref.at[s_idx, :, pl.ds((tb0 + tbl) * 8, 8)])

    start_gather(0, 0)

    @pl.loop(0, N_CHUNKS - 1 if N_CHUNKS % 2 else N_CHUNKS - 2, step=2)
    def _chunk(g2):
        start_gather(g2 + 1, 1)
        process(g2, 0)
        start_gather(g2 + 2, 0)
        process(g2 + 1, 1)

    if N_CHUNKS % 2:
        process(N_CHUNKS - 1, 0)
    else:
        start_gather(N_CHUNKS - 1, 1)
        process(N_CHUNKS - 2, 0)
        process(N_CHUNKS - 1, 1)


@jax.jit
def _emb_ln(ids, table, token_type_emb, ln_gamma, ln_beta):
    mesh = plsc.VectorSubcoreMesh(core_axis_name="c", subcore_axis_name="s",
                                  num_cores=NC, num_subcores=NS)
    kfn = pl.kernel(
        _body,
        out_type=jax.ShapeDtypeStruct((SEQ, 8, 64, 128), jnp.float32),
        mesh=mesh,
        scratch_types=[
            pltpu.VMEM((R_PER_W,), jnp.int32),
            pltpu.VMEM((CHUNK, HIDDEN), jnp.float32),
            pltpu.VMEM((CHUNK, HIDDEN), jnp.float32),
            pltpu.VMEM((2, 8, 8, OPITCH), jnp.float32),
            pltpu.VMEM((HIDDEN,), jnp.float32),
            pltpu.VMEM((HIDDEN,), jnp.float32),
            pltpu.VMEM((HIDDEN,), jnp.float32),
            pltpu.SemaphoreType.DMA,
            pltpu.SemaphoreType.DMA,
        ],
        compiler_params=pltpu.CompilerParams(needs_layout_passes=False,
                                             use_tc_tiling_on_sc=False),
    )
    return kfn(ids, table, token_type_emb, ln_gamma, ln_beta)


def kernel(input_ids, word_emb, token_type_emb, ln_gamma, ln_beta):
    ids = jnp.transpose(input_ids).reshape(N_ROWS).astype(jnp.int32)
    out = _emb_ln(ids, word_emb, token_type_emb, ln_gamma, ln_beta)
    out6 = out.reshape(SEQ, 8, 8, 8, 128)
    res = jnp.transpose(out6, (2, 4, 0, 1, 3))
    return res.reshape(BATCH, SEQ, HIDDEN)


# speedup vs baseline: 1.1126x; 1.1126x over previous
import jax
import jax.numpy as jnp
from jax import lax
from jax.experimental import pallas as pl
from jax.experimental.pallas import tpu as pltpu
from jax.experimental.pallas import tpu_sc as plsc

VOCAB = 1000000
HIDDEN = 64
BATCH = 1024
SEQ = 200
EPS = 1e-05

NC, NS, L = 2, 16, 16
NW = NC * NS
N_ROWS = BATCH * SEQ
R_PER_W = N_ROWS // NW
CHUNK = 256
N_CHUNKS = R_PER_W // CHUNK
PW = 2 * HIDDEN


def _rsqrt16(x):
    i = plsc.bitcast(x, jnp.int32)
    i = jnp.int32(0x5F3759DF) - lax.shift_right_logical(i, 1)
    y = plsc.bitcast(i, jnp.float32)
    h = x * 0.5
    y = y * (1.5 - h * y * y)
    y = y * (1.5 - h * y * y)
    return y


def _body(ids_ref, table_ref, tt_ref, gamma_ref, beta_ref, out_ref,
          idx_v, rows0, rows1, obuf0, obuf1, tv_v, gv_v, bv_v,
          gsem0, gsem1, osem0, osem1):
    wid = lax.axis_index("s") * NC + lax.axis_index("c")
    base = wid * R_PER_W

    pltpu.sync_copy(ids_ref.at[pl.ds(wid * R_PER_W, R_PER_W)], idx_v)
    pltpu.sync_copy(tt_ref.at[0], tv_v)
    pltpu.sync_copy(gamma_ref, gv_v)
    pltpu.sync_copy(beta_ref, bv_v)

    tvs = [tv_v[pl.ds(i * L, L)] for i in range(4)]
    gvs = [gv_v[pl.ds(i * L, L)] for i in range(4)]
    bvs = [bv_v[pl.ds(i * L, L)] for i in range(4)]

    rows = (rows0, rows1)
    gsems = (gsem0, gsem1)
    obufs = (obuf0, obuf1)
    osems = (osem0, osem1)

    def start_gather(g, b):
        idx = idx_v.at[pl.ds(g * CHUNK, CHUNK)]
        pltpu.async_copy(table_ref.at[idx], rows[b], gsems[b])

    def out_dst(g):
        return out_ref.at[pl.ds((base + g * CHUNK) // 2, CHUNK // 2)]

    def process(g, b, owait, pre):
        buf = rows[b]
        obuf = obufs[b]
        idx = idx_v.at[pl.ds(g * CHUNK, CHUNK)]
        pltpu.make_async_copy(table_ref.at[idx], buf, gsems[b]).wait()

        @plsc.parallel_loop(0, CHUNK // L, unroll=2)
        def _grp(j):
            for r in range(L):
                i = j * L + r
                y = [buf[i, pl.ds(k * L, L)] + tvs[k] for k in range(4)]
                s = jnp.sum((y[0] + y[1]) + (y[2] + y[3]))
                q = jnp.sum((y[0] * y[0] + y[1] * y[1])
                            + (y[2] * y[2] + y[3] * y[3]))
                mean = s * (1.0 / HIDDEN)
                var = q * (1.0 / HIDDEN) - mean * mean
                mb = jnp.full((L,), mean, dtype=jnp.float32)
                rb = _rsqrt16(jnp.full((L,), var + EPS, dtype=jnp.float32))
                ip = j * (L // 2) + r // 2
                oc = (r % 2) * HIDDEN
                for k in range(4):
                    obuf[ip, pl.ds(oc + k * L, L)] = ((y[k] - mb) * rb
                                                      * gvs[k] + bvs[k])

        if owait:
            pltpu.make_async_copy(obuf, out_dst(g - 2), osems[b]).wait()
        pltpu.async_copy(obuf, out_dst(g), osems[b])
        if pre is not None:
            start_gather(pre, b)

    start_gather(0, 0)
    start_gather(1, 1)
    process(0, 0, False, 2)
    process(1, 1, False, 3)

    PEEL_START = N_CHUNKS - 2 - (N_CHUNKS % 2)

    @pl.loop(2, PEEL_START, step=2)
    def _chunk(g2):
        process(g2, 0, True, g2 + 2)
        process(g2 + 1, 1, True, g2 + 3)

    for g in range(PEEL_START, N_CHUNKS):
        process(g, g % 2, True, g + 2 if g + 2 < N_CHUNKS else None)

    pltpu.make_async_copy(obufs[(N_CHUNKS - 2) % 2], out_dst(N_CHUNKS - 2),
                          osems[(N_CHUNKS - 2) % 2]).wait()
    pltpu.make_async_copy(obufs[(N_CHUNKS - 1) % 2], out_dst(N_CHUNKS - 1),
                          osems[(N_CHUNKS - 1) % 2]).wait()


@jax.jit
def _emb_ln(ids, table, token_type_emb, ln_gamma, ln_beta):
    mesh = plsc.VectorSubcoreMesh(core_axis_name="c", subcore_axis_name="s",
                                  num_cores=NC, num_subcores=NS)
    kfn = pl.kernel(
        _body,
        out_type=jax.ShapeDtypeStruct((N_ROWS // 2, PW), jnp.float32),
        mesh=mesh,
        scratch_types=[
            pltpu.VMEM((R_PER_W,), jnp.int32),
            pltpu.VMEM((CHUNK, HIDDEN), jnp.float32),
            pltpu.VMEM((CHUNK, HIDDEN), jnp.float32),
            pltpu.VMEM((CHUNK // 2, PW), jnp.float32),
            pltpu.VMEM((CHUNK // 2, PW), jnp.float32),
            pltpu.VMEM((HIDDEN,), jnp.float32),
            pltpu.VMEM((HIDDEN,), jnp.float32),
            pltpu.VMEM((HIDDEN,), jnp.float32),
            pltpu.SemaphoreType.DMA,
            pltpu.SemaphoreType.DMA,
            pltpu.SemaphoreType.DMA,
            pltpu.SemaphoreType.DMA,
        ],
        compiler_params=pltpu.CompilerParams(needs_layout_passes=False,
                                             use_tc_tiling_on_sc=False),
    )
    return kfn(ids, table, token_type_emb, ln_gamma, ln_beta)


def kernel(input_ids, word_emb, token_type_emb, ln_gamma, ln_beta):
    ids = jnp.transpose(input_ids).reshape(N_ROWS).astype(jnp.int32)
    out = _emb_ln(ids, word_emb, token_type_emb, ln_gamma, ln_beta)
    return jnp.transpose(out.reshape(SEQ, BATCH, HIDDEN), (1, 0, 2))


# speedup vs baseline: 1.1676x; 1.0493x over previous
import jax
import jax.numpy as jnp
from jax import lax
from jax.experimental import pallas as pl
from jax.experimental.pallas import tpu as pltpu
from jax.experimental.pallas import tpu_sc as plsc

VOCAB = 1000000
HIDDEN = 64
BATCH = 1024
SEQ = 200
EPS = 1e-05

NC, NS, L = 2, 16, 16
NW = NC * NS
N_ROWS = BATCH * SEQ
R_PER_W = N_ROWS // NW
CHUNK = 256
N_CHUNKS = R_PER_W // CHUNK
PW = 2 * HIDDEN


def _rsqrt16(x):
    i = plsc.bitcast(x, jnp.int32)
    i = jnp.int32(0x5F3759DF) - lax.shift_right_logical(i, 1)
    y = plsc.bitcast(i, jnp.float32)
    h = x * 0.5
    y = y * (1.5 - h * y * y)
    y = y * (1.5 - h * y * y)
    return y


def _body(ids_ref, table_ref, tt_ref, gamma_ref, beta_ref, out_ref,
          idx_v, rows0, rows1, obuf, tv_v, gv_v, bv_v, gsem0, gsem1):
    wid = lax.axis_index("s") * NC + lax.axis_index("c")
    base = wid * R_PER_W

    pltpu.sync_copy(ids_ref.at[pl.ds(wid * R_PER_W, R_PER_W)], idx_v)
    pltpu.sync_copy(tt_ref.at[0], tv_v)
    pltpu.sync_copy(gamma_ref, gv_v)
    pltpu.sync_copy(beta_ref, bv_v)

    tvs = [tv_v[pl.ds(i * L, L)] for i in range(4)]
    gvs = [gv_v[pl.ds(i * L, L)] for i in range(4)]
    bvs = [bv_v[pl.ds(i * L, L)] for i in range(4)]

    rows = (rows0, rows1)
    gsems = (gsem0, gsem1)

    def start_gather(g, b):
        idx = idx_v.at[pl.ds(g * CHUNK, CHUNK)]
        pltpu.async_copy(table_ref.at[idx], rows[b], gsems[b])

    def process(g, b):
        buf = rows[b]
        idx = idx_v.at[pl.ds(g * CHUNK, CHUNK)]
        pltpu.make_async_copy(table_ref.at[idx], buf, gsems[b]).wait()

        @plsc.parallel_loop(0, CHUNK // L, unroll=1)
        def _grp(j):
            for r in range(L):
                i = j * L + r
                y = [buf[i, pl.ds(k * L, L)] + tvs[k] for k in range(4)]
                s = jnp.sum((y[0] + y[1]) + (y[2] + y[3]))
                q = jnp.sum((y[0] * y[0] + y[1] * y[1])
                            + (y[2] * y[2] + y[3] * y[3]))
                mean = s * (1.0 / HIDDEN)
                var = q * (1.0 / HIDDEN) - mean * mean
                mb = jnp.full((L,), mean, dtype=jnp.float32)
                rb = _rsqrt16(jnp.full((L,), var + EPS, dtype=jnp.float32))
                ip = j * (L // 2) + r // 2
                oc = (r % 2) * HIDDEN
                for k in range(4):
                    obuf[ip, pl.ds(oc + k * L, L)] = ((y[k] - mb) * rb
                                                      * gvs[k] + bvs[k])

        pltpu.sync_copy(obuf,
                        out_ref.at[pl.ds((base + g * CHUNK) // 2, CHUNK // 2)])

    start_gather(0, 0)

    @pl.loop(0, N_CHUNKS - 1 if N_CHUNKS % 2 else N_CHUNKS - 2, step=2)
    def _chunk(g2):
        start_gather(g2 + 1, 1)
        process(g2, 0)
        start_gather(g2 + 2, 0)
        process(g2 + 1, 1)

    if N_CHUNKS % 2:
        process(N_CHUNKS - 1, 0)
    else:
        start_gather(N_CHUNKS - 1, 1)
        process(N_CHUNKS - 2, 0)
        process(N_CHUNKS - 1, 1)


@jax.jit
def _emb_ln(ids, table, token_type_emb, ln_gamma, ln_beta):
    mesh = plsc.VectorSubcoreMesh(core_axis_name="c", subcore_axis_name="s",
                                  num_cores=NC, num_subcores=NS)
    kfn = pl.kernel(
        _body,
        out_type=jax.ShapeDtypeStruct((N_ROWS // 2, PW), jnp.float32),
        mesh=mesh,
        scratch_types=[
            pltpu.VMEM((R_PER_W,), jnp.int32),
            pltpu.VMEM((CHUNK, HIDDEN), jnp.float32),
            pltpu.VMEM((CHUNK, HIDDEN), jnp.float32),
            pltpu.VMEM((CHUNK // 2, PW), jnp.float32),
            pltpu.VMEM((HIDDEN,), jnp.float32),
            pltpu.VMEM((HIDDEN,), jnp.float32),
            pltpu.VMEM((HIDDEN,), jnp.float32),
            pltpu.SemaphoreType.DMA,
            pltpu.SemaphoreType.DMA,
        ],
        compiler_params=pltpu.CompilerParams(needs_layout_passes=False,
                                             use_tc_tiling_on_sc=False),
    )
    return kfn(ids, table, token_type_emb, ln_gamma, ln_beta)


def kernel(input_ids, word_emb, token_type_emb, ln_gamma, ln_beta):
    ids = jnp.transpose(input_ids).reshape(N_ROWS).astype(jnp.int32)
    out = _emb_ln(ids, word_emb, token_type_emb, ln_gamma, ln_beta)
    return jnp.transpose(out.reshape(SEQ, BATCH, HIDDEN), (1, 0, 2))
